# baseline (speedup 1.0000x reference)
import functools
import math

import jax
import jax.numpy as jnp
from jax import lax
from jax.experimental import pallas as pl
from jax.experimental.pallas import tpu as pltpu
from jax.experimental.pallas import tpu_sc as plsc

N = 10000
D = 128
E = 320000
C = 40
ALPHA = 0.9
LAMBDA = 1.0

NC = 2
NS = 16
NW = NC * NS
NPAD = 10112
RPT = NPAD // NS
DUMMY = N

CH = 128
EPT = E // NW
CAP = 84
NCHP = CAP
K = 64
FLEN = 10752
NCH = FLEN // K
NBUF = 3


@functools.cache
def _sc_mesh():
    return plsc.VectorSubcoreMesh(core_axis_name="c", subcore_axis_name="s")


@functools.cache
def _sc_deg_fn():
    @functools.partial(
        pl.kernel,
        out_type=jax.ShapeDtypeStruct((NC, NPAD, D), jnp.float32),
        mesh=_sc_mesh(),
        scratch_types=[
            pltpu.VMEM_SHARED((NPAD, D), jnp.float32),
            pltpu.VMEM((CAP, CH), jnp.int32),
            pltpu.VMEM((CH, D), jnp.float32),
            pltpu.SemaphoreType.DMA,
        ],
    )
    def deg_kernel(dstp, ones_h, zeros_h, deg_out, deg_sh, dst_v, ones_v,
                   sem):
        c = lax.axis_index("c")
        s = lax.axis_index("s")
        wid = s * NC + c
        pltpu.sync_copy(zeros_h.at[pl.ds(s * RPT, RPT)],
                        deg_sh.at[pl.ds(s * RPT, RPT)])
        pltpu.sync_copy(dstp.at[wid], dst_v)
        pltpu.sync_copy(ones_h, ones_v)
        plsc.subcore_barrier()

        @pl.loop(0, NCHP)
        def _(j):
            pltpu.async_copy(ones_v, deg_sh.at[dst_v.at[j]], sem, add=True)

        @pl.loop(0, NCHP)
        def _(j):
            pltpu.make_async_copy(ones_v, deg_sh.at[dst_v.at[0]], sem).wait()

        plsc.subcore_barrier()
        pltpu.sync_copy(deg_sh.at[pl.ds(s * RPT, RPT)],
                        deg_out.at[c, pl.ds(s * RPT, RPT)])

    return deg_kernel


@functools.cache
def _sc_layer_fn():
    @functools.partial(
        pl.kernel,
        out_type=jax.ShapeDtypeStruct((NC, NPAD, D), jnp.float32),
        mesh=_sc_mesh(),
        scratch_types=(
            [pltpu.VMEM_SHARED((NPAD, D), jnp.float32),
             pltpu.VMEM((FLEN,), jnp.int32),
             pltpu.VMEM((FLEN,), jnp.int32)]
            + [pltpu.VMEM((K, D), jnp.float32)] * NBUF
            + [pltpu.SemaphoreType.DMA] * (2 * NBUF)
        ),
    )
    def layer_kernel(hs, srcf, dstf, zeros_h, m_out,
                     m_sh, src_v, dst_v, *bufsem):
        bufs = bufsem[:NBUF]
        gsems = bufsem[NBUF:2 * NBUF]
        ssems = bufsem[2 * NBUF:]
        c = lax.axis_index("c")
        s = lax.axis_index("s")
        wid = s * NC + c
        pltpu.sync_copy(zeros_h.at[pl.ds(s * RPT, RPT)],
                        m_sh.at[pl.ds(s * RPT, RPT)])
        pltpu.sync_copy(srcf.at[pl.ds(wid * FLEN, FLEN)], src_v)
        pltpu.sync_copy(dstf.at[pl.ds(wid * FLEN, FLEN)], dst_v)
        plsc.subcore_barrier()

        def gather(ch, b):
            pltpu.async_copy(hs.at[src_v.at[pl.ds(ch * K, K)]],
                             bufs[b], gsems[b])

        def wait_gather(b):
            pltpu.make_async_copy(hs.at[src_v.at[pl.ds(0, K)]],
                                  bufs[b], gsems[b]).wait()

        def scatter(ch, b):
            pltpu.async_copy(bufs[b], m_sh.at[dst_v.at[pl.ds(ch * K, K)]],
                             ssems[b], add=True)

        def wait_scatter(b):
            pltpu.make_async_copy(bufs[b], m_sh.at[dst_v.at[pl.ds(0, K)]],
                                  ssems[b]).wait()

        gather(0, 0)
        for ch in range(2):
            wait_gather(ch)
            scatter(ch, ch)
            gather(ch + 1, ch + 1)

        @pl.loop(0, (NCH - 3) // NBUF)
        def _(g):
            for b0 in range(NBUF):
                ch = 2 + g * NBUF + b0
                b = (2 + b0) % NBUF
                bp = (b + 1) % NBUF
                wait_gather(b)
                scatter(ch, b)
                wait_scatter(bp)
                gather(ch + 1, bp)

        wait_gather((NCH - 1) % NBUF)
        scatter(NCH - 1, (NCH - 1) % NBUF)
        for ch in range(NCH - 3, NCH):
            wait_scatter(ch % NBUF)

        plsc.subcore_barrier()
        pltpu.sync_copy(m_sh.at[pl.ds(s * RPT, RPT)],
                        m_out.at[c, pl.ds(s * RPT, RPT)])

    return layer_kernel


BLK = 1000


def _tc_prep(deg2, x):
    def body(deg_ref, x_ref, norm_ref, hs_ref):
        d = deg_ref[0, :, 0:1] + deg_ref[1, :, 0:1]
        nrm = lax.rsqrt(jnp.maximum(d, 1.0))
        nb = jnp.broadcast_to(nrm, (BLK, D))
        norm_ref[...] = nb
        hs_ref[...] = x_ref[...] * nb

    return pl.pallas_call(
        body,
        grid=(N // BLK,),
        in_specs=[
            pl.BlockSpec((NC, BLK, D), lambda j: (0, j, 0)),
            pl.BlockSpec((BLK, D), lambda j: (j, 0)),
        ],
        out_specs=[
            pl.BlockSpec((BLK, D), lambda j: (j, 0)),
            pl.BlockSpec((BLK, D), lambda j: (j, 0)),
        ],
        out_shape=[
            jax.ShapeDtypeStruct((N, D), jnp.float32),
            jax.ShapeDtypeStruct((NPAD, D), jnp.float32),
        ],
    )(deg2, x)


def _tc_layer(m2, x, normb, W, beta):
    def body(m_ref, x_ref, n_ref, w_ref, hs_ref):
        mcat = m_ref[0] + m_ref[1]
        nb = n_ref[...]
        g = mcat * nb * (1.0 - ALPHA) + ALPHA * x_ref[...]
        hw = jnp.dot(g, w_ref[...], preferred_element_type=jnp.float32)
        h = jnp.maximum((1.0 - beta) * g + beta * hw, 0.0)
        hs_ref[...] = h * nb

    return pl.pallas_call(
        body,
        grid=(N // BLK,),
        in_specs=[
            pl.BlockSpec((NC, BLK, D), lambda j: (0, j, 0)),
            pl.BlockSpec((BLK, D), lambda j: (j, 0)),
            pl.BlockSpec((BLK, D), lambda j: (j, 0)),
            pl.BlockSpec((D, D), lambda j: (0, 0)),
        ],
        out_specs=pl.BlockSpec((BLK, D), lambda j: (j, 0)),
        out_shape=jax.ShapeDtypeStruct((NPAD, D), jnp.float32),
    )(m2, x, normb, W)


def _tc_final(m2, x, normb, W, Wfc, bfc2, beta):
    def body(m_ref, x_ref, n_ref, w_ref, wfc_ref, b_ref, out_ref):
        mcat = m_ref[0] + m_ref[1]
        nb = n_ref[...]
        g = mcat * nb * (1.0 - ALPHA) + ALPHA * x_ref[...]
        hw = jnp.dot(g, w_ref[...], preferred_element_type=jnp.float32)
        h = jnp.maximum((1.0 - beta) * g + beta * hw, 0.0)
        out_ref[...] = (jnp.dot(h, wfc_ref[...],
                                preferred_element_type=jnp.float32)
                        + b_ref[...])

    return pl.pallas_call(
        body,
        grid=(N // BLK,),
        in_specs=[
            pl.BlockSpec((NC, BLK, D), lambda j: (0, j, 0)),
            pl.BlockSpec((BLK, D), lambda j: (j, 0)),
            pl.BlockSpec((BLK, D), lambda j: (j, 0)),
            pl.BlockSpec((D, D), lambda j: (0, 0)),
            pl.BlockSpec((D, C), lambda j: (0, 0)),
            pl.BlockSpec((1, C), lambda j: (0, 0)),
        ],
        out_specs=pl.BlockSpec((BLK, C), lambda j: (j, 0)),
        out_shape=jax.ShapeDtypeStruct((N, C), jnp.float32),
    )(m2, x, normb, W, Wfc, bfc2)


def kernel(x, edge_index, W1, W2, W3, W4, W5, Wfc, bfc):
    src = edge_index[0].astype(jnp.int32)
    dst = edge_index[1].astype(jnp.int32)
    dstp = jnp.pad(dst.reshape(NW, EPT), ((0, 0), (0, CAP * CH - EPT)),
                   constant_values=DUMMY).reshape(NW, CAP, CH)
    srcf = jnp.pad(src.reshape(NW, EPT), ((0, 0), (0, FLEN - EPT)),
                   constant_values=DUMMY).reshape(NW * FLEN)
    dstf = jnp.pad(dst.reshape(NW, EPT), ((0, 0), (0, FLEN - EPT)),
                   constant_values=DUMMY).reshape(NW * FLEN)
    zerosd = jnp.zeros((NPAD, D), jnp.float32)
    onesd = jnp.ones((CH, D), jnp.float32)

    deg2 = _sc_deg_fn()(dstp, onesd, zerosd)
    normb, hs = _tc_prep(deg2, x)

    Ws = (W1, W2, W3, W4, W5)
    for i in range(4):
        beta = math.log(LAMBDA / (i + 1) + 1.0)
        m2 = _sc_layer_fn()(hs, srcf, dstf, zerosd)
        hs = _tc_layer(m2, x, normb, Ws[i], beta)
    beta = math.log(LAMBDA / 5.0 + 1.0)
    m2 = _sc_layer_fn()(hs, srcf, dstf, zerosd)
    return _tc_final(m2, x, normb, Ws[4], Wfc, bfc.reshape(1, C), beta)


# speedup vs baseline: 1.7655x; 1.7655x over previous
import functools
import math

import jax
import jax.numpy as jnp
from jax import lax
from jax.experimental import pallas as pl
from jax.experimental.pallas import tpu as pltpu
from jax.experimental.pallas import tpu_sc as plsc

N = 10000
D = 128
E = 320000
C = 40
ALPHA = 0.9
LAMBDA = 1.0

NC = 2
NS = 16
NW = NC * NS
NPAD = 10112
RPT = NPAD // NS
DUMMY = N

EPT = E // NW
K = 216
FLEN = 10368
NCH = FLEN // K
KD = 288
NCHD = FLEN // KD


@functools.cache
def _sc_mesh():
    return plsc.VectorSubcoreMesh(core_axis_name="c", subcore_axis_name="s")


@functools.cache
def _sc_deg_fn():
    @functools.partial(
        pl.kernel,
        out_type=jax.ShapeDtypeStruct((NC, NPAD, D), jnp.float32),
        mesh=_sc_mesh(),
        scratch_types=[
            pltpu.VMEM_SHARED((NPAD, D), jnp.float32),
            pltpu.VMEM((FLEN,), jnp.int32),
            pltpu.VMEM((KD, D), jnp.float32),
            pltpu.SemaphoreType.DMA,
        ],
    )
    def deg_kernel(dstf, ones_h, zeros_h, deg_out, deg_sh, dst_v, ones_v,
                   sem):
        c = lax.axis_index("c")
        s = lax.axis_index("s")
        wid = s * NC + c
        pltpu.sync_copy(zeros_h.at[pl.ds(s * RPT, RPT)],
                        deg_sh.at[pl.ds(s * RPT, RPT)])
        pltpu.sync_copy(dstf.at[pl.ds(wid * FLEN, FLEN)], dst_v)
        pltpu.sync_copy(ones_h, ones_v)
        plsc.subcore_barrier()

        @pl.loop(0, NCHD)
        def _(j):
            pltpu.async_copy(ones_v, deg_sh.at[dst_v.at[pl.ds(j * KD, KD)]],
                             sem, add=True)

        @pl.loop(0, NCHD)
        def _(j):
            pltpu.make_async_copy(ones_v,
                                  deg_sh.at[dst_v.at[pl.ds(0, KD)]],
                                  sem).wait()

        plsc.subcore_barrier()
        pltpu.sync_copy(deg_sh.at[pl.ds(s * RPT, RPT)],
                        deg_out.at[c, pl.ds(s * RPT, RPT)])

    return deg_kernel


@functools.cache
def _sc_layer_fn():
    @functools.partial(
        pl.kernel,
        out_type=jax.ShapeDtypeStruct((NC, NPAD, D), jnp.float32),
        mesh=_sc_mesh(),
        scratch_types=[
            pltpu.VMEM_SHARED((NPAD, D), jnp.float32),
            pltpu.VMEM((FLEN,), jnp.int32),
            pltpu.VMEM((FLEN,), jnp.int32),
            pltpu.VMEM((K, D), jnp.float32),
            pltpu.SemaphoreType.DMA,
        ],
    )
    def layer_kernel(hs, srcf, dstf, zeros_h, m_out,
                     m_sh, src_v, dst_v, buf, gsem):
        c = lax.axis_index("c")
        s = lax.axis_index("s")
        wid = s * NC + c
        pltpu.sync_copy(zeros_h.at[pl.ds(s * RPT, RPT)],
                        m_sh.at[pl.ds(s * RPT, RPT)])
        pltpu.sync_copy(srcf.at[pl.ds(wid * FLEN, FLEN)], src_v)
        pltpu.sync_copy(dstf.at[pl.ds(wid * FLEN, FLEN)], dst_v)
        plsc.subcore_barrier()

        @pl.loop(0, NCH)
        def _(j):
            pltpu.async_copy(hs.at[src_v.at[pl.ds(j * K, K)]],
                             buf, gsem).wait()
            pltpu.sync_copy(buf, m_sh.at[dst_v.at[pl.ds(j * K, K)]],
                            add=True)

        plsc.subcore_barrier()
        pltpu.sync_copy(m_sh.at[pl.ds(s * RPT, RPT)],
                        m_out.at[c, pl.ds(s * RPT, RPT)])

    return layer_kernel


BLK = 1000


def _tc_prep(deg2, x):
    def body(deg_ref, x_ref, norm_ref, hs_ref):
        d = deg_ref[0, :, 0:1] + deg_ref[1, :, 0:1]
        nrm = lax.rsqrt(jnp.maximum(d, 1.0))
        nb = jnp.broadcast_to(nrm, (BLK, D))
        norm_ref[...] = nb
        hs_ref[...] = x_ref[...] * nb

    return pl.pallas_call(
        body,
        grid=(N // BLK,),
        in_specs=[
            pl.BlockSpec((NC, BLK, D), lambda j: (0, j, 0)),
            pl.BlockSpec((BLK, D), lambda j: (j, 0)),
        ],
        out_specs=[
            pl.BlockSpec((BLK, D), lambda j: (j, 0)),
            pl.BlockSpec((BLK, D), lambda j: (j, 0)),
        ],
        out_shape=[
            jax.ShapeDtypeStruct((N, D), jnp.float32),
            jax.ShapeDtypeStruct((NPAD, D), jnp.float32),
        ],
    )(deg2, x)


def _tc_layer(m2, x, normb, W, beta):
    def body(m_ref, x_ref, n_ref, w_ref, hs_ref):
        mcat = m_ref[0] + m_ref[1]
        nb = n_ref[...]
        g = mcat * nb * (1.0 - ALPHA) + ALPHA * x_ref[...]
        hw = jnp.dot(g, w_ref[...], preferred_element_type=jnp.float32)
        h = jnp.maximum((1.0 - beta) * g + beta * hw, 0.0)
        hs_ref[...] = h * nb

    return pl.pallas_call(
        body,
        grid=(N // BLK,),
        in_specs=[
            pl.BlockSpec((NC, BLK, D), lambda j: (0, j, 0)),
            pl.BlockSpec((BLK, D), lambda j: (j, 0)),
            pl.BlockSpec((BLK, D), lambda j: (j, 0)),
            pl.BlockSpec((D, D), lambda j: (0, 0)),
        ],
        out_specs=pl.BlockSpec((BLK, D), lambda j: (j, 0)),
        out_shape=jax.ShapeDtypeStruct((NPAD, D), jnp.float32),
    )(m2, x, normb, W)


def _tc_final(m2, x, normb, W, Wfc, bfc2, beta):
    def body(m_ref, x_ref, n_ref, w_ref, wfc_ref, b_ref, out_ref):
        mcat = m_ref[0] + m_ref[1]
        nb = n_ref[...]
        g = mcat * nb * (1.0 - ALPHA) + ALPHA * x_ref[...]
        hw = jnp.dot(g, w_ref[...], preferred_element_type=jnp.float32)
        h = jnp.maximum((1.0 - beta) * g + beta * hw, 0.0)
        out_ref[...] = (jnp.dot(h, wfc_ref[...],
                                preferred_element_type=jnp.float32)
                        + b_ref[...])

    return pl.pallas_call(
        body,
        grid=(N // BLK,),
        in_specs=[
            pl.BlockSpec((NC, BLK, D), lambda j: (0, j, 0)),
            pl.BlockSpec((BLK, D), lambda j: (j, 0)),
            pl.BlockSpec((BLK, D), lambda j: (j, 0)),
            pl.BlockSpec((D, D), lambda j: (0, 0)),
            pl.BlockSpec((D, C), lambda j: (0, 0)),
            pl.BlockSpec((1, C), lambda j: (0, 0)),
        ],
        out_specs=pl.BlockSpec((BLK, C), lambda j: (j, 0)),
        out_shape=jax.ShapeDtypeStruct((N, C), jnp.float32),
    )(m2, x, normb, W, Wfc, bfc2)


def kernel(x, edge_index, W1, W2, W3, W4, W5, Wfc, bfc):
    src = edge_index[0].astype(jnp.int32)
    dst = edge_index[1].astype(jnp.int32)
    srcf = jnp.pad(src.reshape(NW, EPT), ((0, 0), (0, FLEN - EPT)),
                   constant_values=DUMMY).reshape(NW * FLEN)
    dstf = jnp.pad(dst.reshape(NW, EPT), ((0, 0), (0, FLEN - EPT)),
                   constant_values=DUMMY).reshape(NW * FLEN)
    zerosd = jnp.zeros((NPAD, D), jnp.float32)
    onesd = jnp.ones((KD, D), jnp.float32)

    deg2 = _sc_deg_fn()(dstf, onesd, zerosd)
    normb, hs = _tc_prep(deg2, x)

    Ws = (W1, W2, W3, W4, W5)
    for i in range(4):
        beta = math.log(LAMBDA / (i + 1) + 1.0)
        m2 = _sc_layer_fn()(hs, srcf, dstf, zerosd)
        hs = _tc_layer(m2, x, normb, Ws[i], beta)
    beta = math.log(LAMBDA / 5.0 + 1.0)
    m2 = _sc_layer_fn()(hs, srcf, dstf, zerosd)
    return _tc_final(m2, x, normb, Ws[4], Wfc, bfc.reshape(1, C), beta)


# speedup vs baseline: 1.8818x; 1.0659x over previous
import functools
import math

import jax
import jax.numpy as jnp
from jax import lax
from jax.experimental import pallas as pl
from jax.experimental.pallas import tpu as pltpu
from jax.experimental.pallas import tpu_sc as plsc

N = 10000
D = 128
E = 320000
C = 40
ALPHA = 0.9
LAMBDA = 1.0

NC = 2
NS = 16
NW = NC * NS
NPAD = 10112
RPT = NPAD // NS
DUMMY = N

EPT = E // NW
K = 96
FLEN = 10368
NCH = FLEN // K
KD = 288
FLEND = 10368
NCHD = FLEND // KD


@functools.cache
def _sc_mesh():
    return plsc.VectorSubcoreMesh(core_axis_name="c", subcore_axis_name="s")


@functools.cache
def _sc_deg_fn():
    @functools.partial(
        pl.kernel,
        out_type=jax.ShapeDtypeStruct((NC, NPAD, D), jnp.float32),
        mesh=_sc_mesh(),
        scratch_types=[
            pltpu.VMEM_SHARED((NPAD, D), jnp.float32),
            pltpu.VMEM((FLEN,), jnp.int32),
            pltpu.VMEM((KD, D), jnp.float32),
            pltpu.SemaphoreType.DMA,
        ],
    )
    def deg_kernel(dstf, ones_h, zeros_h, deg_out, deg_sh, dst_v, ones_v,
                   sem):
        c = lax.axis_index("c")
        s = lax.axis_index("s")
        wid = s * NC + c
        pltpu.sync_copy(zeros_h.at[pl.ds(s * RPT, RPT)],
                        deg_sh.at[pl.ds(s * RPT, RPT)])
        pltpu.sync_copy(dstf.at[pl.ds(wid * FLEN, FLEN)], dst_v)
        pltpu.sync_copy(ones_h, ones_v)
        plsc.subcore_barrier()

        @pl.loop(0, NCHD)
        def _(j):
            pltpu.async_copy(ones_v, deg_sh.at[dst_v.at[pl.ds(j * KD, KD)]],
                             sem, add=True)

        @pl.loop(0, NCHD)
        def _(j):
            pltpu.make_async_copy(ones_v,
                                  deg_sh.at[dst_v.at[pl.ds(0, KD)]],
                                  sem).wait()

        plsc.subcore_barrier()
        pltpu.sync_copy(deg_sh.at[pl.ds(s * RPT, RPT)],
                        deg_out.at[c, pl.ds(s * RPT, RPT)])

    return deg_kernel


@functools.cache
def _sc_layer_fn():
    @functools.partial(
        pl.kernel,
        out_type=jax.ShapeDtypeStruct((NC, NPAD, D), jnp.float32),
        mesh=_sc_mesh(),
        scratch_types=[
            pltpu.VMEM_SHARED((NPAD, D), jnp.float32),
            pltpu.VMEM((FLEN,), jnp.int32),
            pltpu.VMEM((FLEN,), jnp.int32),
            pltpu.VMEM((K, D), jnp.float32),
            pltpu.VMEM((K, D), jnp.float32),
            pltpu.SemaphoreType.DMA,
            pltpu.SemaphoreType.DMA,
        ],
    )
    def layer_kernel(hs, srcf, dstf, zeros_h, m_out,
                     m_sh, src_v, dst_v, buf0, buf1, gs0, gs1):
        bufs = (buf0, buf1)
        gsems = (gs0, gs1)
        c = lax.axis_index("c")
        s = lax.axis_index("s")
        wid = s * NC + c
        pltpu.sync_copy(zeros_h.at[pl.ds(s * RPT, RPT)],
                        m_sh.at[pl.ds(s * RPT, RPT)])
        pltpu.sync_copy(srcf.at[pl.ds(wid * FLEN, FLEN)], src_v)
        pltpu.sync_copy(dstf.at[pl.ds(wid * FLEN, FLEN)], dst_v)
        plsc.subcore_barrier()

        def gather(ch, b):
            pltpu.async_copy(hs.at[src_v.at[pl.ds(ch * K, K)]],
                             bufs[b], gsems[b])

        def wait_gather(b):
            pltpu.make_async_copy(hs.at[src_v.at[pl.ds(0, K)]],
                                  bufs[b], gsems[b]).wait()

        def scatter(ch, b):
            pltpu.sync_copy(bufs[b], m_sh.at[dst_v.at[pl.ds(ch * K, K)]],
                            add=True)

        gather(0, 0)
        gather(1, 1)

        @pl.loop(0, NCH // 2 - 1)
        def _(g):
            for b in range(2):
                ch = 2 * g + b
                wait_gather(b)
                scatter(ch, b)
                gather(ch + 2, b)

        for b in range(2):
            wait_gather(b)
            scatter(NCH - 2 + b, b)

        plsc.subcore_barrier()
        pltpu.sync_copy(m_sh.at[pl.ds(s * RPT, RPT)],
                        m_out.at[c, pl.ds(s * RPT, RPT)])

    return layer_kernel


BLK = 1000


def _tc_prep(deg2, x):
    def body(deg_ref, x_ref, norm_ref, hs_ref):
        d = deg_ref[0, :, 0:1] + deg_ref[1, :, 0:1]
        nrm = lax.rsqrt(jnp.maximum(d, 1.0))
        nb = jnp.broadcast_to(nrm, (BLK, D))
        norm_ref[...] = nb
        hs_ref[...] = x_ref[...] * nb

    return pl.pallas_call(
        body,
        grid=(N // BLK,),
        in_specs=[
            pl.BlockSpec((NC, BLK, D), lambda j: (0, j, 0)),
            pl.BlockSpec((BLK, D), lambda j: (j, 0)),
        ],
        out_specs=[
            pl.BlockSpec((BLK, D), lambda j: (j, 0)),
            pl.BlockSpec((BLK, D), lambda j: (j, 0)),
        ],
        out_shape=[
            jax.ShapeDtypeStruct((N, D), jnp.float32),
            jax.ShapeDtypeStruct((NPAD, D), jnp.float32),
        ],
    )(deg2, x)


def _tc_layer(m2, x, normb, W, beta):
    def body(m_ref, x_ref, n_ref, w_ref, hs_ref):
        mcat = m_ref[0] + m_ref[1]
        nb = n_ref[...]
        g = mcat * nb * (1.0 - ALPHA) + ALPHA * x_ref[...]
        hw = jnp.dot(g, w_ref[...], preferred_element_type=jnp.float32)
        h = jnp.maximum((1.0 - beta) * g + beta * hw, 0.0)
        hs_ref[...] = h * nb

    return pl.pallas_call(
        body,
        grid=(N // BLK,),
        in_specs=[
            pl.BlockSpec((NC, BLK, D), lambda j: (0, j, 0)),
            pl.BlockSpec((BLK, D), lambda j: (j, 0)),
            pl.BlockSpec((BLK, D), lambda j: (j, 0)),
            pl.BlockSpec((D, D), lambda j: (0, 0)),
        ],
        out_specs=pl.BlockSpec((BLK, D), lambda j: (j, 0)),
        out_shape=jax.ShapeDtypeStruct((NPAD, D), jnp.float32),
    )(m2, x, normb, W)


def _tc_final(m2, x, normb, W, Wfc, bfc2, beta):
    def body(m_ref, x_ref, n_ref, w_ref, wfc_ref, b_ref, out_ref):
        mcat = m_ref[0] + m_ref[1]
        nb = n_ref[...]
        g = mcat * nb * (1.0 - ALPHA) + ALPHA * x_ref[...]
        hw = jnp.dot(g, w_ref[...], preferred_element_type=jnp.float32)
        h = jnp.maximum((1.0 - beta) * g + beta * hw, 0.0)
        out_ref[...] = (jnp.dot(h, wfc_ref[...],
                                preferred_element_type=jnp.float32)
                        + b_ref[...])

    return pl.pallas_call(
        body,
        grid=(N // BLK,),
        in_specs=[
            pl.BlockSpec((NC, BLK, D), lambda j: (0, j, 0)),
            pl.BlockSpec((BLK, D), lambda j: (j, 0)),
            pl.BlockSpec((BLK, D), lambda j: (j, 0)),
            pl.BlockSpec((D, D), lambda j: (0, 0)),
            pl.BlockSpec((D, C), lambda j: (0, 0)),
            pl.BlockSpec((1, C), lambda j: (0, 0)),
        ],
        out_specs=pl.BlockSpec((BLK, C), lambda j: (j, 0)),
        out_shape=jax.ShapeDtypeStruct((N, C), jnp.float32),
    )(m2, x, normb, W, Wfc, bfc2)


def kernel(x, edge_index, W1, W2, W3, W4, W5, Wfc, bfc):
    src = edge_index[0].astype(jnp.int32)
    dst = edge_index[1].astype(jnp.int32)
    srcf = jnp.pad(src.reshape(NW, EPT), ((0, 0), (0, FLEN - EPT)),
                   constant_values=DUMMY).reshape(NW * FLEN)
    dstf = jnp.pad(dst.reshape(NW, EPT), ((0, 0), (0, FLEN - EPT)),
                   constant_values=DUMMY).reshape(NW * FLEN)
    zerosd = jnp.zeros((NPAD, D), jnp.float32)
    onesd = jnp.ones((KD, D), jnp.float32)

    deg2 = _sc_deg_fn()(dstf, onesd, zerosd)
    normb, hs = _tc_prep(deg2, x)

    Ws = (W1, W2, W3, W4, W5)
    for i in range(4):
        beta = math.log(LAMBDA / (i + 1) + 1.0)
        m2 = _sc_layer_fn()(hs, srcf, dstf, zerosd)
        hs = _tc_layer(m2, x, normb, Ws[i], beta)
    beta = math.log(LAMBDA / 5.0 + 1.0)
    m2 = _sc_layer_fn()(hs, srcf, dstf, zerosd)
    return _tc_final(m2, x, normb, Ws[4], Wfc, bfc.reshape(1, C), beta)
